# baseline (speedup 1.0000x reference)
import functools

import jax
import jax.numpy as jnp
import numpy as np
from jax.experimental import pallas as pl

_B = 16
_HW = 512
_C = 128
_NCLS = 2
_OCC = 256
_PATCH = 32
_G = 16
_NPTS = _G * _G
_MAP = 64
_NCELL = _MAP * _MAP
_CS = np.float32(3.2 / 64.0)
_MAXH = 39


def _body(p_ref, d_ref, wenc_ref, benc_ref, wdec_ref, cam_ref, ahc_ref,
          aht_ref, u_ref, ut_ref, xsel_ref, zsum_ref, out_ref):
    feats = jnp.dot(p_ref[0], wenc_ref[...],
                    preferred_element_type=jnp.float32, precision=jax.lax.Precision.HIGHEST) + benc_ref[...]

    t = (jnp.dot(ahc_ref[0], d_ref[0, 0], preferred_element_type=jnp.float32, precision=jax.lax.Precision.HIGHEST)
         + jnp.dot(ahc_ref[1], d_ref[0, 1], preferred_element_type=jnp.float32, precision=jax.lax.Precision.HIGHEST)
         + jnp.dot(ahc_ref[2], d_ref[0, 2], preferred_element_type=jnp.float32, precision=jax.lax.Precision.HIGHEST))
    ds2 = jnp.dot(t, aht_ref[...], preferred_element_type=jnp.float32, precision=jax.lax.Precision.HIGHEST)
    ds = jnp.concatenate([ds2[r:r + 1, :] for r in range(_G)], axis=1)

    px = ds * cam_ref[0:1, :]
    py = ds * cam_ref[1:2, :] + 1.0
    pz = ds * cam_ref[2:3, :]
    x_idx = jnp.floor(px / _CS).astype(jnp.int32) + _MAP // 2
    y_idx = jnp.floor(py / _CS).astype(jnp.int32)
    z_idx = jnp.floor(pz / _CS).astype(jnp.int32) + _MAP
    valid = ((x_idx >= 0) & (x_idx < _MAP) & (z_idx >= 0) & (z_idx < _MAP)
             & (y_idx < _MAXH))
    lc = jnp.where(valid, z_idx * _MAP + x_idx, -1)

    cells = jax.lax.broadcasted_iota(jnp.int32, (_NCELL, _NPTS), 0)
    m = lc == cells
    ymat = jnp.where(m, jnp.broadcast_to(y_idx, (_NCELL, _NPTS)), -1)
    maxy = jnp.max(ymat, axis=1, keepdims=True)
    sel = (m & (y_idx == maxy)).astype(jnp.float32)
    warped = jnp.dot(sel, feats, preferred_element_type=jnp.float32, precision=jax.lax.Precision.HIGHEST)

    for n in range(_NCLS):
        yflat = jnp.dot(warped, wdec_ref[:, n:n + 1],
                        preferred_element_type=jnp.float32, precision=jax.lax.Precision.HIGHEST)
        yn = jnp.dot(zsum_ref[...], yflat * xsel_ref[...],
                     preferred_element_type=jnp.float32, precision=jax.lax.Precision.HIGHEST)
        up = jnp.dot(jnp.dot(u_ref[...], yn, preferred_element_type=jnp.float32, precision=jax.lax.Precision.HIGHEST),
                     ut_ref[...], preferred_element_type=jnp.float32, precision=jax.lax.Precision.HIGHEST)
        out_ref[0, n] = up


def kernel(inputs, W_enc, b_enc, W_dec, cam_coords):
    rgb = inputs[:, :3]
    depth = inputs[:, 3:]

    p = rgb.reshape(_B, 3, _G, _PATCH, _G, _PATCH)
    p = p.transpose(0, 2, 4, 1, 3, 5).reshape(_B, _G * _G, 3 * _PATCH * _PATCH)

    ah = jax.image.resize(jnp.eye(_HW, dtype=jnp.float32), (_G, _HW), 'bilinear')
    wc = jax.image.resize(jnp.eye(3, dtype=jnp.float32), (1, 3), 'bilinear')[0]
    ahc = wc[:, None, None] * ah[None]
    u = jax.image.resize(jnp.eye(_MAP, dtype=jnp.float32), (_OCC, _MAP),
                         'bilinear')
    xsel = jnp.asarray(np.tile(np.eye(_MAP, dtype=np.float32), (_MAP, 1)))
    zsum = jnp.asarray(np.repeat(np.eye(_MAP, dtype=np.float32), _MAP, axis=1))

    grid_spec = pl.GridSpec(
        grid=(_B,),
        in_specs=[
            pl.BlockSpec((1, _NPTS, 3 * _PATCH * _PATCH), lambda b: (b, 0, 0)),
            pl.BlockSpec((1, 3, _HW, _HW), lambda b: (b, 0, 0, 0)),
            pl.BlockSpec((3 * _PATCH * _PATCH, _C), lambda b: (0, 0)),
            pl.BlockSpec((1, _C), lambda b: (0, 0)),
            pl.BlockSpec((_C, _NCLS), lambda b: (0, 0)),
            pl.BlockSpec((3, _NPTS), lambda b: (0, 0)),
            pl.BlockSpec((3, _G, _HW), lambda b: (0, 0, 0)),
            pl.BlockSpec((_HW, _G), lambda b: (0, 0)),
            pl.BlockSpec((_OCC, _MAP), lambda b: (0, 0)),
            pl.BlockSpec((_MAP, _OCC), lambda b: (0, 0)),
            pl.BlockSpec((_NCELL, _MAP), lambda b: (0, 0)),
            pl.BlockSpec((_MAP, _NCELL), lambda b: (0, 0)),
        ],
        out_specs=pl.BlockSpec((1, _NCLS, _OCC, _OCC), lambda b: (b, 0, 0, 0)),
    )

    return pl.pallas_call(
        _body,
        grid_spec=grid_spec,
        out_shape=jax.ShapeDtypeStruct((_B, _NCLS, _OCC, _OCC), jnp.float32),
    )(p, depth, W_enc, b_enc.reshape(1, _C), W_dec, cam_coords,
      ahc, ah.T, u, u.T, xsel, zsum)


# speedup vs baseline: 1.3378x; 1.3378x over previous
import jax
import jax.numpy as jnp
import numpy as np
from jax.experimental import pallas as pl

_B = 16
_HW = 512
_C = 128
_NCLS = 2
_OCC = 256
_PATCH = 32
_G = 16
_NPTS = _G * _G
_MAP = 64
_CS = np.float32(3.2 / 64.0)
_MAXH = 39
_HI = jax.lax.Precision.HIGHEST
_NT = (((1,), (1,)), ((), ()))


def _body(p_ref, d_ref, wenc_ref, benc_ref, wdecT_ref, cam_ref, camT_ref,
          ahc_ref, ah_ref, aht_ref, u_ref, ut_ref, out_ref):
    feats = jnp.dot(p_ref[0], wenc_ref[...],
                    preferred_element_type=jnp.float32,
                    precision=_HI) + benc_ref[...]
    gT = jax.lax.dot_general(wdecT_ref[...], feats, _NT,
                             preferred_element_type=jnp.float32,
                             precision=_HI)

    t = (jnp.dot(ahc_ref[0], d_ref[0, 0], preferred_element_type=jnp.float32,
                 precision=_HI)
         + jnp.dot(ahc_ref[1], d_ref[0, 1], preferred_element_type=jnp.float32,
                   precision=_HI)
         + jnp.dot(ahc_ref[2], d_ref[0, 2], preferred_element_type=jnp.float32,
                   precision=_HI))
    ds2 = jnp.dot(t, aht_ref[...], preferred_element_type=jnp.float32,
                  precision=_HI)
    ds2t = jax.lax.dot_general(ah_ref[...], t, _NT,
                               preferred_element_type=jnp.float32,
                               precision=_HI)
    ds_r = jnp.concatenate([ds2[r:r + 1, :] for r in range(_G)], axis=1)
    ds_c = jnp.concatenate([ds2t[:, r:r + 1] for r in range(_G)], axis=0)

    def vox(ds, cx, cy, cz):
        px, py, pz = ds * cx, ds * cy + 1.0, ds * cz
        x = jnp.floor(px / _CS).astype(jnp.int32) + _MAP // 2
        y = jnp.floor(py / _CS).astype(jnp.int32)
        z = jnp.floor(pz / _CS).astype(jnp.int32) + _MAP
        valid = ((x >= 0) & (x < _MAP) & (z >= 0) & (z < _MAP) & (y < _MAXH))
        return x, y, z, valid

    x_r, y_r, z_r, valid_r = vox(ds_r, cam_ref[0:1, :], cam_ref[1:2, :],
                                 cam_ref[2:3, :])
    x_c, y_c, z_c, _ = vox(ds_c, camT_ref[:, 0:1], camT_ref[:, 1:2],
                           camT_ref[:, 2:3])
    lc_r = jnp.where(valid_r, z_r * _MAP + x_r, -1)
    lc_c = z_c * _MAP + x_c

    dom = (lc_c == lc_r) & (y_c > y_r)
    domf = jnp.max(dom.astype(jnp.float32), axis=0, keepdims=True)
    maskf = jnp.where(valid_r & (domf < 0.5), 1.0, 0.0)

    rows = jax.lax.broadcasted_iota(jnp.int32, (_MAP, _NPTS), 0)
    mz = (rows == z_r).astype(jnp.float32)
    mx = (rows == x_r).astype(jnp.float32)

    for n in range(_NCLS):
        s1 = mz * (gT[n:n + 1, :] * maskf)
        yn = jax.lax.dot_general(s1, mx, _NT,
                                 preferred_element_type=jnp.float32,
                                 precision=_HI)
        up = jnp.dot(jnp.dot(u_ref[...], yn, preferred_element_type=jnp.float32,
                             precision=_HI), ut_ref[...],
                     preferred_element_type=jnp.float32, precision=_HI)
        out_ref[0, n] = up


def kernel(inputs, W_enc, b_enc, W_dec, cam_coords):
    rgb = inputs[:, :3]
    depth = inputs[:, 3:]

    p = rgb.reshape(_B, 3, _G, _PATCH, _G, _PATCH)
    p = p.transpose(0, 2, 4, 1, 3, 5).reshape(_B, _G * _G, 3 * _PATCH * _PATCH)

    ah = jax.image.resize(jnp.eye(_HW, dtype=jnp.float32), (_G, _HW), 'bilinear')
    wc = jax.image.resize(jnp.eye(3, dtype=jnp.float32), (1, 3), 'bilinear')[0]
    ahc = wc[:, None, None] * ah[None]
    u = jax.image.resize(jnp.eye(_MAP, dtype=jnp.float32), (_OCC, _MAP),
                         'bilinear')

    grid_spec = pl.GridSpec(
        grid=(_B,),
        in_specs=[
            pl.BlockSpec((1, _NPTS, 3 * _PATCH * _PATCH), lambda b: (b, 0, 0)),
            pl.BlockSpec((1, 3, _HW, _HW), lambda b: (b, 0, 0, 0)),
            pl.BlockSpec((3 * _PATCH * _PATCH, _C), lambda b: (0, 0)),
            pl.BlockSpec((1, _C), lambda b: (0, 0)),
            pl.BlockSpec((_NCLS, _C), lambda b: (0, 0)),
            pl.BlockSpec((3, _NPTS), lambda b: (0, 0)),
            pl.BlockSpec((_NPTS, 3), lambda b: (0, 0)),
            pl.BlockSpec((3, _G, _HW), lambda b: (0, 0, 0)),
            pl.BlockSpec((_G, _HW), lambda b: (0, 0)),
            pl.BlockSpec((_HW, _G), lambda b: (0, 0)),
            pl.BlockSpec((_OCC, _MAP), lambda b: (0, 0)),
            pl.BlockSpec((_MAP, _OCC), lambda b: (0, 0)),
        ],
        out_specs=pl.BlockSpec((1, _NCLS, _OCC, _OCC), lambda b: (b, 0, 0, 0)),
    )

    return pl.pallas_call(
        _body,
        grid_spec=grid_spec,
        out_shape=jax.ShapeDtypeStruct((_B, _NCLS, _OCC, _OCC), jnp.float32),
    )(p, depth, W_enc, b_enc.reshape(1, _C), W_dec.T, cam_coords,
      cam_coords.T, ahc, ah, ah.T, u, u.T)


# speedup vs baseline: 5.7712x; 4.3141x over previous
import jax
import jax.numpy as jnp
import numpy as np
from jax.experimental import pallas as pl

_B = 16
_HW = 512
_NCLS = 2
_OCC = 256
_PATCH = 32
_G = 16
_NPTS = _G * _G
_MAP = 64
_CS = np.float32(3.2 / 64.0)
_MAXH = 39
_HI = jax.lax.Precision.HIGHEST
_NT = (((1,), (1,)), ((), ()))


def _body(in_ref, tk_ref, gb_ref, cam_ref, camT_ref, ahc_ref, ah_ref,
          aht_ref, bones_ref, u_ref, ut_ref, out_ref):
    g_rows = []
    for n in range(_NCLS):
        esum = None
        for c in range(3):
            e = in_ref[0, c] * tk_ref[c, n]
            e3 = e.reshape(_G, _PATCH, _HW)
            s = jnp.sum(e3, axis=1)
            esum = s if esum is None else esum + s
        og = jnp.dot(esum, bones_ref[...],
                     preferred_element_type=jnp.float32, precision=_HI)
        g_flat = jnp.concatenate([og[r:r + 1, :] for r in range(_G)], axis=1)
        g_rows.append(g_flat + gb_ref[0:1, n:n + 1])

    t = (jnp.dot(ahc_ref[0], in_ref[0, 3], preferred_element_type=jnp.float32,
                 precision=_HI)
         + jnp.dot(ahc_ref[1], in_ref[0, 4], preferred_element_type=jnp.float32,
                   precision=_HI)
         + jnp.dot(ahc_ref[2], in_ref[0, 5], preferred_element_type=jnp.float32,
                   precision=_HI))
    ds2 = jnp.dot(t, aht_ref[...], preferred_element_type=jnp.float32,
                  precision=_HI)
    ds2t = jax.lax.dot_general(ah_ref[...], t, _NT,
                               preferred_element_type=jnp.float32,
                               precision=_HI)
    ds_r = jnp.concatenate([ds2[r:r + 1, :] for r in range(_G)], axis=1)
    ds_c = jnp.concatenate([ds2t[:, r:r + 1] for r in range(_G)], axis=0)

    def vox(ds, cx, cy, cz):
        px, py, pz = ds * cx, ds * cy + 1.0, ds * cz
        x = jnp.floor(px / _CS).astype(jnp.int32) + _MAP // 2
        y = jnp.floor(py / _CS).astype(jnp.int32)
        z = jnp.floor(pz / _CS).astype(jnp.int32) + _MAP
        valid = ((x >= 0) & (x < _MAP) & (z >= 0) & (z < _MAP) & (y < _MAXH))
        return x, y, z, valid

    x_r, y_r, z_r, valid_r = vox(ds_r, cam_ref[0:1, :], cam_ref[1:2, :],
                                 cam_ref[2:3, :])
    x_c, y_c, z_c, _ = vox(ds_c, camT_ref[:, 0:1], camT_ref[:, 1:2],
                           camT_ref[:, 2:3])
    lc_r = jnp.where(valid_r, z_r * _MAP + x_r, -1)
    lc_c = z_c * _MAP + x_c

    dom = (lc_c == lc_r) & (y_c > y_r)
    domf = jnp.max(dom.astype(jnp.float32), axis=0, keepdims=True)
    maskf = jnp.where(valid_r & (domf < 0.5), 1.0, 0.0)

    rows = jax.lax.broadcasted_iota(jnp.int32, (_MAP, _NPTS), 0)
    mz = (rows == z_r).astype(jnp.float32)
    mx = (rows == x_r).astype(jnp.float32)

    for n in range(_NCLS):
        s1 = mz * (g_rows[n] * maskf)
        yn = jax.lax.dot_general(s1, mx, _NT,
                                 preferred_element_type=jnp.float32,
                                 precision=_HI)
        up = jnp.dot(jnp.dot(u_ref[...], yn, preferred_element_type=jnp.float32,
                             precision=_HI), ut_ref[...],
                     preferred_element_type=jnp.float32, precision=_HI)
        out_ref[0, n] = up


def kernel(inputs, W_enc, b_enc, W_dec, cam_coords):
    v = jnp.dot(W_enc, W_dec, preferred_element_type=jnp.float32,
                precision=_HI)
    gbias = jnp.dot(b_enc[None, :], W_dec,
                    preferred_element_type=jnp.float32, precision=_HI)
    v4 = v.reshape(3, _PATCH, _PATCH, _NCLS)
    tk = jnp.tile(v4.transpose(0, 3, 1, 2), (1, 1, _G, _G))

    ah = jax.image.resize(jnp.eye(_HW, dtype=jnp.float32), (_G, _HW), 'bilinear')
    wc = jax.image.resize(jnp.eye(3, dtype=jnp.float32), (1, 3), 'bilinear')[0]
    ahc = wc[:, None, None] * ah[None]
    u = jax.image.resize(jnp.eye(_MAP, dtype=jnp.float32), (_OCC, _MAP),
                         'bilinear')
    bones = jnp.asarray(np.repeat(np.eye(_G, dtype=np.float32), _PATCH,
                                  axis=0))

    grid_spec = pl.GridSpec(
        grid=(_B,),
        in_specs=[
            pl.BlockSpec((1, 6, _HW, _HW), lambda b: (b, 0, 0, 0)),
            pl.BlockSpec((3, _NCLS, _HW, _HW), lambda b: (0, 0, 0, 0)),
            pl.BlockSpec((1, _NCLS), lambda b: (0, 0)),
            pl.BlockSpec((3, _NPTS), lambda b: (0, 0)),
            pl.BlockSpec((_NPTS, 3), lambda b: (0, 0)),
            pl.BlockSpec((3, _G, _HW), lambda b: (0, 0, 0)),
            pl.BlockSpec((_G, _HW), lambda b: (0, 0)),
            pl.BlockSpec((_HW, _G), lambda b: (0, 0)),
            pl.BlockSpec((_HW, _G), lambda b: (0, 0)),
            pl.BlockSpec((_OCC, _MAP), lambda b: (0, 0)),
            pl.BlockSpec((_MAP, _OCC), lambda b: (0, 0)),
        ],
        out_specs=pl.BlockSpec((1, _NCLS, _OCC, _OCC), lambda b: (b, 0, 0, 0)),
    )

    return pl.pallas_call(
        _body,
        grid_spec=grid_spec,
        out_shape=jax.ShapeDtypeStruct((_B, _NCLS, _OCC, _OCC), jnp.float32),
    )(inputs, tk, gbias, cam_coords, cam_coords.T, ahc, ah, ah.T, bones,
      u, u.T)


# speedup vs baseline: 6.7847x; 1.1756x over previous
import jax
import jax.numpy as jnp
import numpy as np
from jax.experimental import pallas as pl

_B = 16
_HW = 512
_NCLS = 2
_OCC = 256
_PATCH = 32
_G = 16
_NPTS = _G * _G
_MAP = 64
_CS = np.float32(3.2 / 64.0)
_MAXH = 39
_HI = jax.lax.Precision.HIGHEST
_NT = (((1,), (1,)), ((), ()))
_BPB = 2


def _body(in_ref, tk_ref, gb_ref, cam_ref, camT_ref, ahc_ref, ah_ref,
          aht_ref, bones_ref, u_ref, ut_ref, out_ref):
  for b in range(_BPB):
    g_rows = []
    for n in range(_NCLS):
        esum = None
        for c in range(3):
            e = in_ref[b, c] * tk_ref[c, n]
            e3 = e.reshape(_G, _PATCH, _HW)
            s = jnp.sum(e3, axis=1)
            esum = s if esum is None else esum + s
        og = jnp.dot(esum, bones_ref[...],
                     preferred_element_type=jnp.float32)
        g_flat = jnp.concatenate([og[r:r + 1, :] for r in range(_G)], axis=1)
        g_rows.append(g_flat + gb_ref[0:1, n:n + 1])

    t = (jnp.dot(ahc_ref[0], in_ref[b, 3], preferred_element_type=jnp.float32,
                 precision=_HI)
         + jnp.dot(ahc_ref[1], in_ref[b, 4], preferred_element_type=jnp.float32,
                   precision=_HI)
         + jnp.dot(ahc_ref[2], in_ref[b, 5], preferred_element_type=jnp.float32,
                   precision=_HI))
    ds2 = jnp.dot(t, aht_ref[...], preferred_element_type=jnp.float32,
                  precision=_HI)
    ds2t = jax.lax.dot_general(ah_ref[...], t, _NT,
                               preferred_element_type=jnp.float32,
                               precision=_HI)
    ds_r = jnp.concatenate([ds2[r:r + 1, :] for r in range(_G)], axis=1)
    ds_c = jnp.concatenate([ds2t[:, r:r + 1] for r in range(_G)], axis=0)

    def vox(ds, cx, cy, cz):
        px, py, pz = ds * cx, ds * cy + 1.0, ds * cz
        x = jnp.floor(px / _CS).astype(jnp.int32) + _MAP // 2
        y = jnp.floor(py / _CS).astype(jnp.int32)
        z = jnp.floor(pz / _CS).astype(jnp.int32) + _MAP
        valid = ((x >= 0) & (x < _MAP) & (z >= 0) & (z < _MAP) & (y < _MAXH))
        return x, y, z, valid

    x_r, y_r, z_r, valid_r = vox(ds_r, cam_ref[0:1, :], cam_ref[1:2, :],
                                 cam_ref[2:3, :])
    x_c, y_c, z_c, _ = vox(ds_c, camT_ref[:, 0:1], camT_ref[:, 1:2],
                           camT_ref[:, 2:3])
    lc_r = jnp.where(valid_r, z_r * _MAP + x_r, -1)
    lc_c = z_c * _MAP + x_c

    dom = (lc_c == lc_r) & (y_c > y_r)
    domf = jnp.max(dom.astype(jnp.float32), axis=0, keepdims=True)
    maskf = jnp.where(valid_r & (domf < 0.5), 1.0, 0.0)

    rows = jax.lax.broadcasted_iota(jnp.int32, (_MAP, _NPTS), 0)
    mz = (rows == z_r).astype(jnp.float32)
    mx = (rows == x_r).astype(jnp.float32)

    for n in range(_NCLS):
        s1 = mz * (g_rows[n] * maskf)
        yn = jax.lax.dot_general(s1, mx, _NT,
                                 preferred_element_type=jnp.float32)
        up = jnp.dot(jnp.dot(u_ref[...], yn,
                             preferred_element_type=jnp.float32),
                     ut_ref[...], preferred_element_type=jnp.float32)
        out_ref[b, n] = up


def kernel(inputs, W_enc, b_enc, W_dec, cam_coords):
    v = jnp.dot(W_enc, W_dec, preferred_element_type=jnp.float32,
                precision=_HI)
    gbias = jnp.dot(b_enc[None, :], W_dec,
                    preferred_element_type=jnp.float32, precision=_HI)
    v4 = v.reshape(3, _PATCH, _PATCH, _NCLS)
    tk = jnp.tile(v4.transpose(0, 3, 1, 2), (1, 1, _G, _G))

    ah = jax.image.resize(jnp.eye(_HW, dtype=jnp.float32), (_G, _HW), 'bilinear')
    wc = jax.image.resize(jnp.eye(3, dtype=jnp.float32), (1, 3), 'bilinear')[0]
    ahc = wc[:, None, None] * ah[None]
    u = jax.image.resize(jnp.eye(_MAP, dtype=jnp.float32), (_OCC, _MAP),
                         'bilinear')
    bones = jnp.asarray(np.repeat(np.eye(_G, dtype=np.float32), _PATCH,
                                  axis=0))

    grid_spec = pl.GridSpec(
        grid=(_B // _BPB,),
        in_specs=[
            pl.BlockSpec((_BPB, 6, _HW, _HW), lambda b: (b, 0, 0, 0)),
            pl.BlockSpec((3, _NCLS, _HW, _HW), lambda b: (0, 0, 0, 0)),
            pl.BlockSpec((1, _NCLS), lambda b: (0, 0)),
            pl.BlockSpec((3, _NPTS), lambda b: (0, 0)),
            pl.BlockSpec((_NPTS, 3), lambda b: (0, 0)),
            pl.BlockSpec((3, _G, _HW), lambda b: (0, 0, 0)),
            pl.BlockSpec((_G, _HW), lambda b: (0, 0)),
            pl.BlockSpec((_HW, _G), lambda b: (0, 0)),
            pl.BlockSpec((_HW, _G), lambda b: (0, 0)),
            pl.BlockSpec((_OCC, _MAP), lambda b: (0, 0)),
            pl.BlockSpec((_MAP, _OCC), lambda b: (0, 0)),
        ],
        out_specs=pl.BlockSpec((_BPB, _NCLS, _OCC, _OCC), lambda b: (b, 0, 0, 0)),
    )

    return pl.pallas_call(
        _body,
        grid_spec=grid_spec,
        out_shape=jax.ShapeDtypeStruct((_B, _NCLS, _OCC, _OCC), jnp.float32),
    )(inputs, tk, gbias, cam_coords, cam_coords.T, ahc, ah, ah.T, bones,
      u, u.T)


# speedup vs baseline: 6.7870x; 1.0003x over previous
import jax
import jax.numpy as jnp
import numpy as np
from jax.experimental import pallas as pl

_B = 16
_HW = 512
_NCLS = 2
_OCC = 256
_PATCH = 32
_G = 16
_NPTS = _G * _G
_MAP = 64
_CS = np.float32(3.2 / 64.0)
_MAXH = 39
_HI = jax.lax.Precision.HIGHEST
_NT = (((1,), (1,)), ((), ()))
_BPB = 2


def _body(in_ref, dep_ref, tk_ref, gb_ref, cam_ref, camT_ref, ahc_ref, ah_ref,
          aht_ref, bones_ref, u_ref, ut_ref, out_ref):
  for b in range(_BPB):
    g_rows = []
    for n in range(_NCLS):
        esum = None
        for c in range(3):
            e = in_ref[b, c] * tk_ref[c, n]
            e3 = e.reshape(_G, _PATCH, _HW)
            s = jnp.sum(e3, axis=1)
            esum = s if esum is None else esum + s
        og = jnp.dot(esum, bones_ref[...],
                     preferred_element_type=jnp.float32)
        g_flat = jnp.concatenate([og[r:r + 1, :] for r in range(_G)], axis=1)
        g_rows.append(g_flat + gb_ref[0:1, n:n + 1])

    t = (jnp.dot(ahc_ref[0], dep_ref[b, 0], preferred_element_type=jnp.float32,
                 precision=_HI)
         + jnp.dot(ahc_ref[1], dep_ref[b, 1], preferred_element_type=jnp.float32,
                   precision=_HI)
         + jnp.dot(ahc_ref[2], dep_ref[b, 2], preferred_element_type=jnp.float32,
                   precision=_HI))
    ds2 = jnp.dot(t, aht_ref[...], preferred_element_type=jnp.float32,
                  precision=_HI)
    ds2t = jax.lax.dot_general(ah_ref[...], t, _NT,
                               preferred_element_type=jnp.float32,
                               precision=_HI)
    ds_r = jnp.concatenate([ds2[r:r + 1, :] for r in range(_G)], axis=1)
    ds_c = jnp.concatenate([ds2t[:, r:r + 1] for r in range(_G)], axis=0)

    def vox(ds, cx, cy, cz):
        px, py, pz = ds * cx, ds * cy + 1.0, ds * cz
        x = jnp.floor(px / _CS).astype(jnp.int32) + _MAP // 2
        y = jnp.floor(py / _CS).astype(jnp.int32)
        z = jnp.floor(pz / _CS).astype(jnp.int32) + _MAP
        valid = ((x >= 0) & (x < _MAP) & (z >= 0) & (z < _MAP) & (y < _MAXH))
        return x, y, z, valid

    x_r, y_r, z_r, valid_r = vox(ds_r, cam_ref[0:1, :], cam_ref[1:2, :],
                                 cam_ref[2:3, :])
    x_c, y_c, z_c, _ = vox(ds_c, camT_ref[:, 0:1], camT_ref[:, 1:2],
                           camT_ref[:, 2:3])
    lc_r = jnp.where(valid_r, z_r * _MAP + x_r, -1)
    lc_c = z_c * _MAP + x_c

    dom = (lc_c == lc_r) & (y_c > y_r)
    domf = jnp.max(dom.astype(jnp.float32), axis=0, keepdims=True)
    maskf = jnp.where(valid_r & (domf < 0.5), 1.0, 0.0)

    rows = jax.lax.broadcasted_iota(jnp.int32, (_MAP, _NPTS), 0)
    mz = (rows == z_r).astype(jnp.float32)
    mx = (rows == x_r).astype(jnp.float32)

    for n in range(_NCLS):
        s1 = mz * (g_rows[n] * maskf)
        yn = jax.lax.dot_general(s1, mx, _NT,
                                 preferred_element_type=jnp.float32)
        up = jnp.dot(jnp.dot(u_ref[...], yn,
                             preferred_element_type=jnp.float32),
                     ut_ref[...], preferred_element_type=jnp.float32)
        out_ref[b, n] = up


def kernel(inputs, W_enc, b_enc, W_dec, cam_coords):
    v = jnp.dot(W_enc, W_dec, preferred_element_type=jnp.float32,
                precision=_HI)
    gbias = jnp.dot(b_enc[None, :], W_dec,
                    preferred_element_type=jnp.float32, precision=_HI)
    v4 = v.reshape(3, _PATCH, _PATCH, _NCLS)
    tk = jnp.tile(v4.transpose(0, 3, 1, 2), (1, 1, _G, _G))

    ah = jax.image.resize(jnp.eye(_HW, dtype=jnp.float32), (_G, _HW), 'bilinear')
    wc = jax.image.resize(jnp.eye(3, dtype=jnp.float32), (1, 3), 'bilinear')[0]
    ahc = wc[:, None, None] * ah[None]
    u = jax.image.resize(jnp.eye(_MAP, dtype=jnp.float32), (_OCC, _MAP),
                         'bilinear')
    bones = jnp.asarray(np.repeat(np.eye(_G, dtype=np.float32), _PATCH,
                                  axis=0))

    grid_spec = pl.GridSpec(
        grid=(_B // _BPB,),
        in_specs=[
            pl.BlockSpec((_BPB, 3, _HW, _HW), lambda b: (b, 0, 0, 0)),
            pl.BlockSpec((_BPB, 3, _HW, _HW), lambda b: (b, 1, 0, 0)),
            pl.BlockSpec((3, _NCLS, _HW, _HW), lambda b: (0, 0, 0, 0)),
            pl.BlockSpec((1, _NCLS), lambda b: (0, 0)),
            pl.BlockSpec((3, _NPTS), lambda b: (0, 0)),
            pl.BlockSpec((_NPTS, 3), lambda b: (0, 0)),
            pl.BlockSpec((3, _G, _HW), lambda b: (0, 0, 0)),
            pl.BlockSpec((_G, _HW), lambda b: (0, 0)),
            pl.BlockSpec((_HW, _G), lambda b: (0, 0)),
            pl.BlockSpec((_HW, _G), lambda b: (0, 0)),
            pl.BlockSpec((_OCC, _MAP), lambda b: (0, 0)),
            pl.BlockSpec((_MAP, _OCC), lambda b: (0, 0)),
        ],
        out_specs=pl.BlockSpec((_BPB, _NCLS, _OCC, _OCC), lambda b: (b, 0, 0, 0)),
    )

    return pl.pallas_call(
        _body,
        grid_spec=grid_spec,
        out_shape=jax.ShapeDtypeStruct((_B, _NCLS, _OCC, _OCC), jnp.float32),
    )(inputs, inputs, tk, gbias, cam_coords, cam_coords.T, ahc, ah, ah.T,
      bones, u, u.T)


# speedup vs baseline: 6.7951x; 1.0012x over previous
import jax
import jax.numpy as jnp
import numpy as np
from jax.experimental import pallas as pl

_B = 16
_HW = 512
_NCLS = 2
_OCC = 256
_PATCH = 32
_G = 16
_NPTS = _G * _G
_MAP = 64
_CS = np.float32(3.2 / 64.0)
_MAXH = 39
_HI = jax.lax.Precision.HIGHEST
_NT = (((1,), (1,)), ((), ()))
_BPB = 2


def _body(in_ref, tk_ref, gb_ref, cam_ref, camT_ref, ahc_ref, ah_ref,
          aht_ref, bones_ref, u_ref, ut_ref, out_ref):
  for b in range(_BPB):
    g_rows = []
    for n in range(_NCLS):
        esum = None
        for c in range(3):
            e = in_ref[b, c] * tk_ref[c, n]
            e3 = e.reshape(_G, _PATCH, _HW)
            s = jnp.sum(e3, axis=1)
            esum = s if esum is None else esum + s
        og = jnp.dot(esum, bones_ref[...],
                     preferred_element_type=jnp.float32)
        g_flat = jnp.concatenate([og[r:r + 1, :] for r in range(_G)], axis=1)
        g_rows.append(g_flat + gb_ref[0:1, n:n + 1])

    t = (jnp.dot(ahc_ref[0], in_ref[b, 3], preferred_element_type=jnp.float32,
                 precision=_HI)
         + jnp.dot(ahc_ref[1], in_ref[b, 4], preferred_element_type=jnp.float32,
                   precision=_HI)
         + jnp.dot(ahc_ref[2], in_ref[b, 5], preferred_element_type=jnp.float32,
                   precision=_HI))
    ds2 = jnp.dot(t, aht_ref[...], preferred_element_type=jnp.float32,
                  precision=_HI)
    ds2t = jax.lax.dot_general(ah_ref[...], t, _NT,
                               preferred_element_type=jnp.float32,
                               precision=_HI)
    ds_r = jnp.concatenate([ds2[r:r + 1, :] for r in range(_G)], axis=1)
    ds_c = jnp.concatenate([ds2t[:, r:r + 1] for r in range(_G)], axis=0)

    def vox(ds, cx, cy, cz):
        px, py, pz = ds * cx, ds * cy + 1.0, ds * cz
        x = jnp.floor(px / _CS).astype(jnp.int32) + _MAP // 2
        y = jnp.floor(py / _CS).astype(jnp.int32)
        z = jnp.floor(pz / _CS).astype(jnp.int32) + _MAP
        valid = ((x >= 0) & (x < _MAP) & (z >= 0) & (z < _MAP) & (y < _MAXH))
        return x, y, z, valid

    x_r, y_r, z_r, valid_r = vox(ds_r, cam_ref[0:1, :], cam_ref[1:2, :],
                                 cam_ref[2:3, :])
    x_c, y_c, z_c, _ = vox(ds_c, camT_ref[:, 0:1], camT_ref[:, 1:2],
                           camT_ref[:, 2:3])
    lc_r = jnp.where(valid_r, z_r * _MAP + x_r, -1)
    lc_c = z_c * _MAP + x_c

    dom = (lc_c == lc_r) & (y_c > y_r)
    domf = jnp.max(dom.astype(jnp.float32), axis=0, keepdims=True)
    maskf = jnp.where(valid_r & (domf < 0.5), 1.0, 0.0)

    rows = jax.lax.broadcasted_iota(jnp.int32, (_MAP, _NPTS), 0)
    mz = (rows == z_r).astype(jnp.float32)
    mx = (rows == x_r).astype(jnp.float32)

    for n in range(_NCLS):
        s1 = mz * (g_rows[n] * maskf)
        yn = jax.lax.dot_general(s1, mx, _NT,
                                 preferred_element_type=jnp.float32)
        up = jnp.dot(jnp.dot(u_ref[...], yn,
                             preferred_element_type=jnp.float32),
                     ut_ref[...], preferred_element_type=jnp.float32)
        out_ref[b, n] = up


def kernel(inputs, W_enc, b_enc, W_dec, cam_coords):
    v = jnp.dot(W_enc, W_dec, preferred_element_type=jnp.float32,
                precision=_HI)
    gbias = jnp.dot(b_enc[None, :], W_dec,
                    preferred_element_type=jnp.float32, precision=_HI)
    v4 = v.reshape(3, _PATCH, _PATCH, _NCLS)
    tk = jnp.tile(v4.transpose(0, 3, 1, 2), (1, 1, _G, _G))

    ah = jax.image.resize(jnp.eye(_HW, dtype=jnp.float32), (_G, _HW), 'bilinear')
    wc = jax.image.resize(jnp.eye(3, dtype=jnp.float32), (1, 3), 'bilinear')[0]
    ahc = wc[:, None, None] * ah[None]
    u = jax.image.resize(jnp.eye(_MAP, dtype=jnp.float32), (_OCC, _MAP),
                         'bilinear')
    bones = jnp.asarray(np.repeat(np.eye(_G, dtype=np.float32), _PATCH,
                                  axis=0))

    grid_spec = pl.GridSpec(
        grid=(_B // _BPB,),
        in_specs=[
            pl.BlockSpec((_BPB, 6, _HW, _HW), lambda b: (b, 0, 0, 0)),
            pl.BlockSpec((3, _NCLS, _HW, _HW), lambda b: (0, 0, 0, 0)),
            pl.BlockSpec((1, _NCLS), lambda b: (0, 0)),
            pl.BlockSpec((3, _NPTS), lambda b: (0, 0)),
            pl.BlockSpec((_NPTS, 3), lambda b: (0, 0)),
            pl.BlockSpec((3, _G, _HW), lambda b: (0, 0, 0)),
            pl.BlockSpec((_G, _HW), lambda b: (0, 0)),
            pl.BlockSpec((_HW, _G), lambda b: (0, 0)),
            pl.BlockSpec((_HW, _G), lambda b: (0, 0)),
            pl.BlockSpec((_OCC, _MAP), lambda b: (0, 0)),
            pl.BlockSpec((_MAP, _OCC), lambda b: (0, 0)),
        ],
        out_specs=pl.BlockSpec((_BPB, _NCLS, _OCC, _OCC), lambda b: (b, 0, 0, 0)),
    )

    return pl.pallas_call(
        _body,
        grid_spec=grid_spec,
        out_shape=jax.ShapeDtypeStruct((_B, _NCLS, _OCC, _OCC), jnp.float32),
    )(inputs, tk, gbias, cam_coords, cam_coords.T, ahc, ah, ah.T, bones,
      u, u.T)
